# baseline (speedup 1.0000x reference)
import functools

import jax
import jax.numpy as jnp
from jax import lax
from jax.experimental import pallas as pl
from jax.experimental.pallas import tpu as pltpu
from jax.experimental.pallas import tpu_sc as plsc

_N = 6
_D = 32
_L = 16
_STEPS = _N - 1


def _energy_body(x_ref, w1_ref, b1_ref, w2_ref, b2_ref, w3_ref, b3_ref, o_ref):
    x = x_ref[...]
    h = jnp.dot(x, w1_ref[...], preferred_element_type=jnp.float32)
    h = jnp.maximum(h + b1_ref[...], 0.0)
    h = jnp.dot(h, w2_ref[...], preferred_element_type=jnp.float32)
    h = jnp.maximum(h + b2_ref[...], 0.0)
    z = jnp.dot(h, w3_ref[...], preferred_element_type=jnp.float32)
    z = z + b3_ref[...]
    o_ref[...] = jnp.maximum(z, 0.0) + jnp.log(1.0 + jnp.exp(-jnp.abs(z)))


def _energy_head(x8, w1t, b1r, w2t, b2r, w3t, b3r):
    return pl.pallas_call(
        _energy_body,
        out_shape=jax.ShapeDtypeStruct((8, 8), jnp.float32),
    )(x8, w1t, b1r, w2t, b2r, w3t, b3r)


def _lane_bcast(vec, lane):
    idx = jnp.full((_L,), lane, dtype=jnp.int32)
    return jnp.take_along_axis(vec, idx, axis=0, mode="promise_in_bounds")


def _sample_body(xa_hbm, xb_hbm, wpt_hbm, bp_hbm, g_hbm, out_hbm,
                 xa_v, xb_v, wpt_v, bp_v, g_v, l_v, path_v):
    cid = lax.axis_index("c")
    sid = lax.axis_index("s")

    @pl.when(jnp.logical_and(cid == 0, sid == 0))
    def _():
        pltpu.sync_copy(xa_hbm, xa_v)
        pltpu.sync_copy(xb_hbm, xb_v)
        pltpu.sync_copy(wpt_hbm, wpt_v)
        pltpu.sync_copy(bp_hbm, bp_v)
        pltpu.sync_copy(g_hbm, g_v)

        bp_vec = bp_v[...]
        rows = []
        for i in range(_N):
            xia = xa_v[i, :]
            xib = xb_v[i, :]
            acc = bp_vec
            for k in range(_D):
                xk = _lane_bcast(xia if k < _L else xib, k % _L)
                acc = acc + xk * wpt_v[k, :]
            l_v[i, :] = acc
            rows.append(acc)

        iota = lax.iota(jnp.int32, _L)
        alive = jnp.where((iota >= 1) & (iota < _N), 1, 0)
        path_vec = jnp.zeros((_L,), jnp.int32)
        lcur = rows[0]
        for t in range(_STEPS):
            alive_f = alive.astype(jnp.float32)
            pos = (lax.cumsum(alive_f, axis=0) - alive_f).astype(jnp.int32)
            g = plsc.load_gather(
                g_v, [jnp.full((_L,), t, jnp.int32), pos])
            score = jnp.where(alive == 1, lcur + g, -1e30)
            best = jnp.max(score)
            chosen = jnp.min(jnp.where(score == best, iota, 2 * _L))
            chosen_vec = jnp.broadcast_to(chosen, (_L,))
            path_vec = jnp.where(iota == t + 1, chosen_vec, path_vec)
            alive = jnp.where(iota == chosen_vec, 0, alive)
            if t + 1 < _STEPS:
                lcur = plsc.load_gather(l_v, [chosen_vec, iota])
        path_v[...] = path_vec
        pltpu.sync_copy(path_v, out_hbm)


_sample_kernel = functools.partial(
    pl.kernel,
    out_type=jax.ShapeDtypeStruct((_L,), jnp.int32),
    mesh=plsc.VectorSubcoreMesh(core_axis_name="c", subcore_axis_name="s"),
    compiler_params=pltpu.CompilerParams(needs_layout_passes=False),
    scratch_types=[
        pltpu.VMEM((_N, _L), jnp.float32),
        pltpu.VMEM((_N, _L), jnp.float32),
        pltpu.VMEM((_D, _L), jnp.float32),
        pltpu.VMEM((_L,), jnp.float32),
        pltpu.VMEM((_STEPS, _L), jnp.float32),
        pltpu.VMEM((_N, _L), jnp.float32),
        pltpu.VMEM((_L,), jnp.int32),
    ],
)(_sample_body)


def _gumbel_table():
    key = jax.random.key(42)
    rows = []
    for m in range(_N - 1, 0, -1):
        key, sk = jax.random.split(key)
        g = jax.random.gumbel(sk, (m,), jnp.float32)
        rows.append(jnp.pad(g, (0, _L - m)))
    return jnp.stack(rows)


def kernel(x, path, W1, b1, W2, b2, W3, b3, Wp, bp):
    del path

    x8 = jnp.zeros((8, _D), jnp.float32).at[:_N].set(x)
    w1t = W1.T
    w2t = W2.T
    w3t = jnp.zeros((_D, 8), jnp.float32).at[:, :1].set(W3.T)
    b1r = b1.reshape(1, 64)
    b2r = b2.reshape(1, 32)
    b3r = jnp.zeros((1, 8), jnp.float32).at[0, 0].set(b3[0])
    energy8 = _energy_head(x8, w1t, b1r, w2t, b2r, w3t, b3r)
    energy = energy8[:_N, 0]

    xa = x[:, :_L]
    xb = x[:, _L:]
    wpt = jnp.zeros((_D, _L), jnp.float32).at[:, :_N].set(Wp.T)
    bp16 = jnp.pad(bp, (0, _L - _N))
    gtab = _gumbel_table()
    path16 = _sample_kernel(xa, xb, wpt, bp16, gtab)
    path_indices = path16[:_N + 1]

    return (energy, path_indices)


# speedup vs baseline: 1.0216x; 1.0216x over previous
import functools

import jax
import jax.numpy as jnp
from jax import lax
from jax.experimental import pallas as pl
from jax.experimental.pallas import tpu as pltpu
from jax.experimental.pallas import tpu_sc as plsc

_N = 6
_D = 32
_L = 16
_STEPS = _N - 1


def _energy_body(x_ref, w1_ref, b1_ref, w2_ref, b2_ref, w3_ref, b3_ref, o_ref):
    x = x_ref[...]
    h = jnp.dot(x, w1_ref[...], preferred_element_type=jnp.float32)
    h = jnp.maximum(h + b1_ref[...], 0.0)
    h = jnp.dot(h, w2_ref[...], preferred_element_type=jnp.float32)
    h = jnp.maximum(h + b2_ref[...], 0.0)
    z = jnp.dot(h, w3_ref[...], preferred_element_type=jnp.float32)
    z = z + b3_ref[...]
    o_ref[...] = jnp.maximum(z, 0.0) + jnp.log(1.0 + jnp.exp(-jnp.abs(z)))


def _energy_head(x8, w1t, b1r, w2t, b2r, w3t, b3r):
    return pl.pallas_call(
        _energy_body,
        out_shape=jax.ShapeDtypeStruct((8, 8), jnp.float32),
    )(x8, w1t, b1r, w2t, b2r, w3t, b3r)


def _lane_bcast(vec, lane):
    idx = jnp.full((_L,), lane, dtype=jnp.int32)
    return jnp.take_along_axis(vec, idx, axis=0, mode="promise_in_bounds")


def _sample_body(xa_hbm, xb_hbm, wpt_hbm, bp_hbm, g_hbm, out_hbm,
                 xa_v, xb_v, wpt_v, bp_v, g_v, l_v, path_v):
    cid = lax.axis_index("c")
    sid = lax.axis_index("s")

    @pl.when(jnp.logical_and(cid == 0, sid == 0))
    def _():
        pltpu.sync_copy(xa_hbm, xa_v)
        pltpu.sync_copy(xb_hbm, xb_v)
        pltpu.sync_copy(wpt_hbm, wpt_v)
        pltpu.sync_copy(bp_hbm, bp_v)
        pltpu.sync_copy(g_hbm, g_v)

        bp_vec = bp_v[...]
        rows = []
        for i in range(_N):
            xia = xa_v[i, :]
            xib = xb_v[i, :]
            acc = bp_vec
            for k in range(_D):
                xk = _lane_bcast(xia if k < _L else xib, k % _L)
                acc = acc + xk * wpt_v[k, :]
            l_v[i, :] = acc
            rows.append(acc)

        iota = lax.iota(jnp.int32, _L)
        alive = jnp.where((iota >= 1) & (iota < _N), 1, 0)
        path_vec = jnp.zeros((_L,), jnp.int32)
        lcur = rows[0]
        for t in range(_STEPS):
            alive_f = alive.astype(jnp.float32)
            pos = (lax.cumsum(alive_f, axis=0) - alive_f).astype(jnp.int32)
            g = plsc.load_gather(
                g_v, [jnp.full((_L,), t, jnp.int32), pos])
            score = jnp.where(alive == 1, lcur + g, -1e30)
            best = jnp.max(score)
            chosen = jnp.min(jnp.where(score == best, iota, 2 * _L))
            chosen_vec = jnp.broadcast_to(chosen, (_L,))
            path_vec = jnp.where(iota == t + 1, chosen_vec, path_vec)
            alive = jnp.where(iota == chosen_vec, 0, alive)
            if t + 1 < _STEPS:
                lcur = plsc.load_gather(l_v, [chosen_vec, iota])
        path_v[...] = path_vec
        pltpu.sync_copy(path_v, out_hbm)


_sample_kernel = functools.partial(
    pl.kernel,
    out_type=jax.ShapeDtypeStruct((_L,), jnp.int32),
    mesh=plsc.VectorSubcoreMesh(
        core_axis_name="c", subcore_axis_name="s", num_cores=1),
    compiler_params=pltpu.CompilerParams(needs_layout_passes=False),
    scratch_types=[
        pltpu.VMEM((_N, _L), jnp.float32),
        pltpu.VMEM((_N, _L), jnp.float32),
        pltpu.VMEM((_D, _L), jnp.float32),
        pltpu.VMEM((_L,), jnp.float32),
        pltpu.VMEM((_STEPS, _L), jnp.float32),
        pltpu.VMEM((_N, _L), jnp.float32),
        pltpu.VMEM((_L,), jnp.int32),
    ],
)(_sample_body)


def _gumbel_table():
    key = jax.random.key(42)
    rows = []
    for m in range(_N - 1, 0, -1):
        key, sk = jax.random.split(key)
        g = jax.random.gumbel(sk, (m,), jnp.float32)
        rows.append(jnp.pad(g, (0, _L - m)))
    return jnp.stack(rows)


def kernel(x, path, W1, b1, W2, b2, W3, b3, Wp, bp):
    del path

    x8 = jnp.zeros((8, _D), jnp.float32).at[:_N].set(x)
    w1t = W1.T
    w2t = W2.T
    w3t = jnp.zeros((_D, 8), jnp.float32).at[:, :1].set(W3.T)
    b1r = b1.reshape(1, 64)
    b2r = b2.reshape(1, 32)
    b3r = jnp.zeros((1, 8), jnp.float32).at[0, 0].set(b3[0])
    energy8 = _energy_head(x8, w1t, b1r, w2t, b2r, w3t, b3r)
    energy = energy8[:_N, 0]

    xa = x[:, :_L]
    xb = x[:, _L:]
    wpt = jnp.zeros((_D, _L), jnp.float32).at[:, :_N].set(Wp.T)
    bp16 = jnp.pad(bp, (0, _L - _N))
    gtab = _gumbel_table()
    path16 = _sample_kernel(xa, xb, wpt, bp16, gtab)
    path_indices = path16[:_N + 1]

    return (energy, path_indices)


# speedup vs baseline: 1.0231x; 1.0015x over previous
import functools

import jax
import jax.numpy as jnp
from jax import lax
from jax.experimental import pallas as pl
from jax.experimental.pallas import tpu as pltpu
from jax.experimental.pallas import tpu_sc as plsc

_N = 6
_D = 32
_L = 16
_STEPS = _N - 1


def _energy_body(x_ref, w1_ref, b1_ref, w2_ref, b2_ref, w3_ref, b3_ref, o_ref):
    x = x_ref[...]
    h = jnp.dot(x, w1_ref[...], preferred_element_type=jnp.float32)
    h = jnp.maximum(h + b1_ref[...], 0.0)
    h = jnp.dot(h, w2_ref[...], preferred_element_type=jnp.float32)
    h = jnp.maximum(h + b2_ref[...], 0.0)
    z = jnp.dot(h, w3_ref[...], preferred_element_type=jnp.float32)
    z = z + b3_ref[...]
    o_ref[...] = jnp.maximum(z, 0.0) + jnp.log(1.0 + jnp.exp(-jnp.abs(z)))


def _energy_head(x8, w1t, b1r, w2t, b2r, w3t, b3r):
    return pl.pallas_call(
        _energy_body,
        out_shape=jax.ShapeDtypeStruct((8, 8), jnp.float32),
    )(x8, w1t, b1r, w2t, b2r, w3t, b3r)


def _lane_bcast(vec, lane):
    idx = jnp.full((_L,), lane, dtype=jnp.int32)
    return jnp.take_along_axis(vec, idx, axis=0, mode="promise_in_bounds")


def _sample_body(xa_hbm, xb_hbm, wpt_hbm, bp_hbm, g_hbm, out_hbm,
                 xa_v, xb_v, wpt_v, bp_v, g_v, l_v, path_v):
    cid = lax.axis_index("c")
    sid = lax.axis_index("s")

    @pl.when(jnp.logical_and(cid == 0, sid == 0))
    def _():
        pltpu.sync_copy(xa_hbm, xa_v)
        pltpu.sync_copy(xb_hbm, xb_v)
        pltpu.sync_copy(wpt_hbm, wpt_v)
        pltpu.sync_copy(bp_hbm, bp_v)
        pltpu.sync_copy(g_hbm, g_v)

        bp_vec = bp_v[...]
        rows = []
        for i in range(_N):
            xia = xa_v[i, :]
            xib = xb_v[i, :]
            acc = bp_vec
            for k in range(_D):
                xk = _lane_bcast(xia if k < _L else xib, k % _L)
                acc = acc + xk * wpt_v[k, :]
            l_v[i, :] = acc
            rows.append(acc)

        iota = lax.iota(jnp.int32, _L)
        alive = jnp.where((iota >= 1) & (iota < _N), 1, 0)
        path_vec = jnp.zeros((_L,), jnp.int32)
        lcur = rows[0]
        for t in range(_STEPS):
            alive_f = alive.astype(jnp.float32)
            pos = (lax.cumsum(alive_f, axis=0) - alive_f).astype(jnp.int32)
            g = plsc.load_gather(
                g_v, [jnp.full((_L,), t, jnp.int32), pos])
            score = jnp.where(alive == 1, lcur + g, -1e30)
            best = jnp.max(score)
            chosen = jnp.min(jnp.where(score == best, iota, 2 * _L))
            chosen_vec = jnp.broadcast_to(chosen, (_L,))
            path_vec = jnp.where(iota == t + 1, chosen_vec, path_vec)
            alive = jnp.where(iota == chosen_vec, 0, alive)
            if t + 1 < _STEPS:
                lcur = plsc.load_gather(l_v, [chosen_vec, iota])
        path_v[...] = path_vec
        pltpu.sync_copy(path_v, out_hbm)


_sample_kernel = functools.partial(
    pl.kernel,
    out_type=jax.ShapeDtypeStruct((_L,), jnp.int32),
    mesh=plsc.VectorSubcoreMesh(
        core_axis_name="c", subcore_axis_name="s",
        num_cores=1, num_subcores=1),
    compiler_params=pltpu.CompilerParams(
        needs_layout_passes=False, skip_device_barrier=True),
    scratch_types=[
        pltpu.VMEM((_N, _L), jnp.float32),
        pltpu.VMEM((_N, _L), jnp.float32),
        pltpu.VMEM((_D, _L), jnp.float32),
        pltpu.VMEM((_L,), jnp.float32),
        pltpu.VMEM((_STEPS, _L), jnp.float32),
        pltpu.VMEM((_N, _L), jnp.float32),
        pltpu.VMEM((_L,), jnp.int32),
    ],
)(_sample_body)


def _gumbel_table():
    key = jax.random.key(42)
    rows = []
    for m in range(_N - 1, 0, -1):
        key, sk = jax.random.split(key)
        g = jax.random.gumbel(sk, (m,), jnp.float32)
        rows.append(jnp.pad(g, (0, _L - m)))
    return jnp.stack(rows)


def kernel(x, path, W1, b1, W2, b2, W3, b3, Wp, bp):
    del path

    x8 = jnp.zeros((8, _D), jnp.float32).at[:_N].set(x)
    w1t = W1.T
    w2t = W2.T
    w3t = jnp.zeros((_D, 8), jnp.float32).at[:, :1].set(W3.T)
    b1r = b1.reshape(1, 64)
    b2r = b2.reshape(1, 32)
    b3r = jnp.zeros((1, 8), jnp.float32).at[0, 0].set(b3[0])
    energy8 = _energy_head(x8, w1t, b1r, w2t, b2r, w3t, b3r)
    energy = energy8[:_N, 0]

    xa = x[:, :_L]
    xb = x[:, _L:]
    wpt = jnp.zeros((_D, _L), jnp.float32).at[:, :_N].set(Wp.T)
    bp16 = jnp.pad(bp, (0, _L - _N))
    gtab = _gumbel_table()
    path16 = _sample_kernel(xa, xb, wpt, bp16, gtab)
    path_indices = path16[:_N + 1]

    return (energy, path_indices)
